# baseline (speedup 1.0000x reference)
import functools
import jax
import jax.numpy as jnp
from jax import lax
from jax.experimental import pallas as pl
from jax.experimental.pallas import tpu as pltpu
from jax.experimental.pallas import tpu_sc as plsc

N = 10000
E = 320000
D_IN = 128
D_HID = 64

NTILE = 16
CHK = 128
TPB = 160
NCH = NTILE * TPB
EP = NCH * CHK
NBUF = 2
IBR = 8
ZR = N // NTILE
NACC = N + 128
NBLK = 25
BR = N // NBLK


def _deg_body(eidx, wch, zn, deg_out, deg_s, idx_v, w_v):
    c = lax.axis_index("c")
    s = lax.axis_index("s")

    @pl.when(s == 0)
    def _():
        pltpu.sync_copy(zn, deg_s)

    plsc.subcore_barrier()
    pltpu.sync_copy(eidx.at[c, s], idx_v)
    pltpu.sync_copy(wch.at[s], w_v)

    @pl.loop(0, TPB)
    def _(j):
        pltpu.sync_copy(w_v.at[j], deg_s.at[idx_v.at[j]], add=True)

    plsc.subcore_barrier()

    @pl.when(s == 0)
    def _():
        pltpu.sync_copy(deg_s, deg_out.at[c])


def _diff_body(tab, gidx, sidx, zb, out, acc, gi_v, si_v, buf, gsem):
    c = lax.axis_index("c")
    s = lax.axis_index("s")
    pltpu.sync_copy(zb, acc.at[pl.ds(s * ZR, ZR)])
    plsc.subcore_barrier()

    @pl.loop(0, TPB // IBR)
    def _(kb):
        base = kb * IBR
        pltpu.sync_copy(gidx.at[c, s, pl.ds(base, IBR)], gi_v)
        pltpu.sync_copy(sidx.at[c, s, pl.ds(base, IBR)], si_v)
        @pl.loop(0, IBR)
        def _(j2):
            pltpu.sync_copy(tab.at[gi_v.at[j2]], buf.at[0])
            pltpu.sync_copy(buf.at[0], acc.at[si_v.at[j2]], add=True)

    plsc.subcore_barrier()
    pltpu.sync_copy(acc.at[pl.ds(s * ZR, ZR)], out.at[c, s])


def _mm_body(x_ref, deg_ref, wp_ref, wq_ref, wa_ref, ba_ref, tab_ref, xa_ref):
    xb = x_ref[...]
    d = deg_ref[0]
    r_out = jnp.reciprocal(d[:, 0:1])
    r_in = jnp.reciprocal(d[:, 1:2])
    hi = lax.Precision.HIGHEST
    tab_ref[0] = jnp.dot(xb, wp_ref[...], precision=hi, preferred_element_type=jnp.float32) * r_out
    tab_ref[1] = jnp.dot(xb, wq_ref[...], precision=hi, preferred_element_type=jnp.float32) * r_in
    xa_ref[...] = jnp.dot(xb, wa_ref[...], precision=hi, preferred_element_type=jnp.float32) + ba_ref[...]


def _fin_body(xa_ref, s_ref, wl_ref, bl_ref, o_ref):
    xa = xa_ref[...]
    s_o = s_ref[0]
    s_i = s_ref[1]
    zp = xa[:, :D_HID] + s_o[:, :D_HID] + s_i[:, :D_HID]
    hp = xa[:, D_HID:] + s_o[:, D_HID:] + s_i[:, D_HID:]
    hnew = (1.0 - jax.nn.sigmoid(zp)) * jnp.tanh(hp)
    o_ref[...] = jnp.dot(hnew, wl_ref[...], precision=lax.Precision.HIGHEST, preferred_element_type=jnp.float32) + bl_ref[...]


def kernel(x, edge_index, edge_weight, Wz, bz, Wr, br, Wh, bh, Wl, bl):
    f32 = jnp.float32
    ei = edge_index.astype(jnp.int32)
    pad = EP - E
    row = jnp.concatenate([ei[0], jnp.zeros((pad,), jnp.int32)])
    col = jnp.concatenate([ei[1], jnp.zeros((pad,), jnp.int32)])
    garb = N + (jnp.arange(pad, dtype=jnp.int32) % 128)
    srow = jnp.concatenate([ei[0], garb])
    scol = jnp.concatenate([ei[1], garb])
    eidx = jnp.stack([row, col]).reshape(2, NTILE, TPB, CHK)
    gidx = jnp.stack([row, col + N]).reshape(2, NTILE, TPB, CHK)
    sidx = jnp.stack([scol, srow]).reshape(2, NTILE, TPB, CHK)
    wch = jnp.concatenate([edge_weight.astype(f32), jnp.zeros((pad,), f32)]).reshape(
        NTILE, TPB, CHK
    )
    zn = jnp.zeros((N,), f32)
    zb = jnp.zeros((ZR, D_IN), f32)

    W_P = jnp.concatenate([Wz[0, 1, :D_IN, :], Wh[0, 1, :D_IN, :]], axis=1)
    W_Q = jnp.concatenate([Wz[1, 1, :D_IN, :], Wh[1, 1, :D_IN, :]], axis=1)
    W_A = jnp.concatenate(
        [Wz[0, 0, :D_IN] + Wz[1, 0, :D_IN], Wh[0, 0, :D_IN] + Wh[1, 0, :D_IN]], axis=1
    )
    b_A = jnp.concatenate([bz, bh]).reshape(1, 2 * D_HID)

    mesh = plsc.VectorSubcoreMesh(core_axis_name="c", subcore_axis_name="s")

    deg = pl.kernel(
        _deg_body,
        out_type=jax.ShapeDtypeStruct((2, N), f32),
        mesh=mesh,
        scratch_types=[
            pltpu.VMEM_SHARED((N,), f32),
            pltpu.VMEM((TPB, CHK), jnp.int32),
            pltpu.VMEM((TPB, CHK), f32),
        ],
        name="dcrnn_degrees",
    )(eidx, wch, zn)
    deg_t = deg.T.reshape(NBLK, BR, 2)

    tab3, xa = pl.pallas_call(
        _mm_body,
        grid=(NBLK,),
        in_specs=[
            pl.BlockSpec((BR, D_IN), lambda i: (i, 0)),
            pl.BlockSpec((1, BR, 2), lambda i: (i, 0, 0)),
            pl.BlockSpec((D_IN, D_IN), lambda i: (0, 0)),
            pl.BlockSpec((D_IN, D_IN), lambda i: (0, 0)),
            pl.BlockSpec((D_IN, D_IN), lambda i: (0, 0)),
            pl.BlockSpec((1, 2 * D_HID), lambda i: (0, 0)),
        ],
        out_specs=[
            pl.BlockSpec((2, BR, D_IN), lambda i: (0, i, 0)),
            pl.BlockSpec((BR, D_IN), lambda i: (i, 0)),
        ],
        out_shape=[
            jax.ShapeDtypeStruct((2, N, D_IN), f32),
            jax.ShapeDtypeStruct((N, D_IN), f32),
        ],
        name="dcrnn_tables",
    )(x, deg_t, W_P, W_Q, W_A, b_A)

    tab = tab3.reshape(2 * N, D_IN)

    s_acc = pl.kernel(
        _diff_body,
        out_type=jax.ShapeDtypeStruct((2, NTILE, ZR, D_IN), f32),
        mesh=mesh,
        scratch_types=[
            pltpu.VMEM_SHARED((NACC, D_IN), f32),
            pltpu.VMEM((IBR, CHK), jnp.int32),
            pltpu.VMEM((IBR, CHK), jnp.int32),
            pltpu.VMEM((NBUF, CHK, D_IN), f32),
            pltpu.SemaphoreType.DMA,
        ],
        name="dcrnn_diffusion",
    )(tab, gidx, sidx, zb)
    s_acc = s_acc.reshape(2, N, D_IN)

    out = pl.pallas_call(
        _fin_body,
        grid=(NBLK,),
        in_specs=[
            pl.BlockSpec((BR, D_IN), lambda i: (i, 0)),
            pl.BlockSpec((2, BR, D_IN), lambda i: (0, i, 0)),
            pl.BlockSpec((D_HID, 1), lambda i: (0, 0)),
            pl.BlockSpec((1, 1), lambda i: (0, 0)),
        ],
        out_specs=pl.BlockSpec((BR, 1), lambda i: (i, 0)),
        out_shape=jax.ShapeDtypeStruct((N, 1), f32),
        name="dcrnn_head",
    )(xa, s_acc, Wl, bl.reshape(1, 1))

    return out[:, 0]


# speedup vs baseline: 1.7189x; 1.7189x over previous
import functools
import jax
import jax.numpy as jnp
from jax import lax
from jax.experimental import pallas as pl
from jax.experimental.pallas import tpu as pltpu
from jax.experimental.pallas import tpu_sc as plsc

N = 10000
E = 320000
D_IN = 128
D_HID = 64

NTILE = 16
CHK = 80
NCH = E // CHK
TPB = NCH // NTILE
NBUF = 2
IBR = 50
ZR = N // NTILE
NBLK = 25
BR = N // NBLK


def _deg_body(eidx, wch, zn, deg_out, deg_s, idx_v, w_v):
    c = lax.axis_index("c")
    s = lax.axis_index("s")

    @pl.when(s == 0)
    def _():
        pltpu.sync_copy(zn, deg_s)

    plsc.subcore_barrier()
    pltpu.sync_copy(eidx.at[c, s], idx_v)
    pltpu.sync_copy(wch.at[s], w_v)

    @pl.loop(0, TPB)
    def _(j):
        pltpu.sync_copy(w_v.at[j], deg_s.at[idx_v.at[j]], add=True)

    plsc.subcore_barrier()

    @pl.when(s == 0)
    def _():
        pltpu.sync_copy(deg_s, deg_out.at[c])


def _diff_body(tab, gidx, sidx, zb, out, acc, gi_v, si_v, buf, gsem):
    c = lax.axis_index("c")
    s = lax.axis_index("s")
    pltpu.sync_copy(zb, acc.at[pl.ds(s * ZR, ZR)])
    plsc.subcore_barrier()

    @pl.loop(0, TPB // IBR)
    def _(kb):
        pltpu.sync_copy(gidx.at[c, s, kb], gi_v)
        pltpu.sync_copy(sidx.at[c, s, kb], si_v)
        @pl.loop(0, IBR // 2)
        def _(p):
            j0 = 2 * p
            j1 = j0 + 1
            pltpu.async_copy(tab.at[gi_v.at[j0]], buf.at[0], gsem)
            pltpu.make_async_copy(tab.at[gi_v.at[j0]], buf.at[0], gsem).wait()
            pltpu.async_copy(tab.at[gi_v.at[j1]], buf.at[1], gsem)
            pltpu.sync_copy(buf.at[0], acc.at[si_v.at[j0]], add=True)
            pltpu.make_async_copy(tab.at[gi_v.at[j1]], buf.at[1], gsem).wait()
            pltpu.sync_copy(buf.at[1], acc.at[si_v.at[j1]], add=True)

    plsc.subcore_barrier()
    pltpu.sync_copy(acc.at[pl.ds(s * ZR, ZR)], out.at[c, s])


def _mm_body(x_ref, deg_ref, wp_ref, wq_ref, wa_ref, ba_ref, tab_ref, xa_ref):
    xb = x_ref[...]
    d = deg_ref[0]
    r_out = jnp.reciprocal(d[:, 0:1])
    r_in = jnp.reciprocal(d[:, 1:2])
    hi = lax.Precision.HIGHEST
    tab_ref[0] = jnp.dot(xb, wp_ref[...], precision=hi, preferred_element_type=jnp.float32) * r_out
    tab_ref[1] = jnp.dot(xb, wq_ref[...], precision=hi, preferred_element_type=jnp.float32) * r_in
    xa_ref[...] = jnp.dot(xb, wa_ref[...], precision=hi, preferred_element_type=jnp.float32) + ba_ref[...]


def _fin_body(xa_ref, s_ref, wl_ref, bl_ref, o_ref):
    xa = xa_ref[...]
    s_o = s_ref[0]
    s_i = s_ref[1]
    zp = xa[:, :D_HID] + s_o[:, :D_HID] + s_i[:, :D_HID]
    hp = xa[:, D_HID:] + s_o[:, D_HID:] + s_i[:, D_HID:]
    hnew = (1.0 - jax.nn.sigmoid(zp)) * jnp.tanh(hp)
    o_ref[...] = jnp.dot(hnew, wl_ref[...], precision=lax.Precision.HIGHEST, preferred_element_type=jnp.float32) + bl_ref[...]


def kernel(x, edge_index, edge_weight, Wz, bz, Wr, br, Wh, bh, Wl, bl):
    f32 = jnp.float32
    ei = edge_index.astype(jnp.int32)
    row, col = ei[0], ei[1]
    eidx = ei.reshape(2, NTILE, TPB, CHK)
    gidx = jnp.stack([row, col + N]).reshape(2, NTILE, TPB // IBR, IBR, CHK)
    sidx = jnp.stack([col, row]).reshape(2, NTILE, TPB // IBR, IBR, CHK)
    wch = edge_weight.astype(f32).reshape(NTILE, TPB, CHK)
    zn = jnp.zeros((N,), f32)
    zb = jnp.zeros((ZR, D_IN), f32)

    W_P = jnp.concatenate([Wz[0, 1, :D_IN, :], Wh[0, 1, :D_IN, :]], axis=1)
    W_Q = jnp.concatenate([Wz[1, 1, :D_IN, :], Wh[1, 1, :D_IN, :]], axis=1)
    W_A = jnp.concatenate(
        [Wz[0, 0, :D_IN] + Wz[1, 0, :D_IN], Wh[0, 0, :D_IN] + Wh[1, 0, :D_IN]], axis=1
    )
    b_A = jnp.concatenate([bz, bh]).reshape(1, 2 * D_HID)

    mesh = plsc.VectorSubcoreMesh(core_axis_name="c", subcore_axis_name="s")

    deg = pl.kernel(
        _deg_body,
        out_type=jax.ShapeDtypeStruct((2, N), f32),
        mesh=mesh,
        scratch_types=[
            pltpu.VMEM_SHARED((N,), f32),
            pltpu.VMEM((TPB, CHK), jnp.int32),
            pltpu.VMEM((TPB, CHK), f32),
        ],
        name="dcrnn_degrees",
    )(eidx, wch, zn)
    deg_t = deg.T.reshape(NBLK, BR, 2)

    tab3, xa = pl.pallas_call(
        _mm_body,
        grid=(NBLK,),
        in_specs=[
            pl.BlockSpec((BR, D_IN), lambda i: (i, 0)),
            pl.BlockSpec((1, BR, 2), lambda i: (i, 0, 0)),
            pl.BlockSpec((D_IN, D_IN), lambda i: (0, 0)),
            pl.BlockSpec((D_IN, D_IN), lambda i: (0, 0)),
            pl.BlockSpec((D_IN, D_IN), lambda i: (0, 0)),
            pl.BlockSpec((1, 2 * D_HID), lambda i: (0, 0)),
        ],
        out_specs=[
            pl.BlockSpec((2, BR, D_IN), lambda i: (0, i, 0)),
            pl.BlockSpec((BR, D_IN), lambda i: (i, 0)),
        ],
        out_shape=[
            jax.ShapeDtypeStruct((2, N, D_IN), f32),
            jax.ShapeDtypeStruct((N, D_IN), f32),
        ],
        name="dcrnn_tables",
    )(x, deg_t, W_P, W_Q, W_A, b_A)

    tab = tab3.reshape(2 * N, D_IN)

    s_acc = pl.kernel(
        _diff_body,
        out_type=jax.ShapeDtypeStruct((2, NTILE, ZR, D_IN), f32),
        mesh=mesh,
        scratch_types=[
            pltpu.VMEM_SHARED((N, D_IN), f32),
            pltpu.VMEM((IBR, CHK), jnp.int32),
            pltpu.VMEM((IBR, CHK), jnp.int32),
            pltpu.VMEM((NBUF, CHK, D_IN), f32),
            pltpu.SemaphoreType.DMA,
        ],
        name="dcrnn_diffusion",
    )(tab, gidx, sidx, zb)
    s_acc = s_acc.reshape(2, N, D_IN)

    out = pl.pallas_call(
        _fin_body,
        grid=(NBLK,),
        in_specs=[
            pl.BlockSpec((BR, D_IN), lambda i: (i, 0)),
            pl.BlockSpec((2, BR, D_IN), lambda i: (0, i, 0)),
            pl.BlockSpec((D_HID, 1), lambda i: (0, 0)),
            pl.BlockSpec((1, 1), lambda i: (0, 0)),
        ],
        out_specs=pl.BlockSpec((BR, 1), lambda i: (i, 0)),
        out_shape=jax.ShapeDtypeStruct((N, 1), f32),
        name="dcrnn_head",
    )(xa, s_acc, Wl, bl.reshape(1, 1))

    return out[:, 0]


# speedup vs baseline: 1.9102x; 1.1113x over previous
import functools
import jax
import jax.numpy as jnp
from jax import lax
from jax.experimental import pallas as pl
from jax.experimental.pallas import tpu as pltpu
from jax.experimental.pallas import tpu_sc as plsc

N = 10000
E = 320000
D_IN = 128
D_HID = 64

NTILE = 16
CHK = 80
NCH = E // CHK
TPB = NCH // NTILE
NBUF = 2
IBR = 50
ZR = N // NTILE
NBLK = 25
BR = N // NBLK


def _deg_body(eidx, wch, zn, deg_out, deg_s, idx_v, w_v):
    c = lax.axis_index("c")
    s = lax.axis_index("s")

    @pl.when(s == 0)
    def _():
        pltpu.sync_copy(zn, deg_s)

    plsc.subcore_barrier()
    pltpu.sync_copy(eidx.at[c, s], idx_v)
    pltpu.sync_copy(wch.at[s], w_v)

    @pl.loop(0, TPB)
    def _(j):
        pltpu.sync_copy(w_v.at[j], deg_s.at[idx_v.at[j]], add=True)

    plsc.subcore_barrier()

    @pl.when(s == 0)
    def _():
        pltpu.sync_copy(deg_s, deg_out.at[c])


def _diff_body(tab, gidx, sidx, zb, out, acc, gi_v, si_v, buf, gsem):
    c = lax.axis_index("c")
    s = lax.axis_index("s")
    pltpu.sync_copy(zb, acc.at[pl.ds(s * ZR, ZR)])
    plsc.subcore_barrier()

    @pl.loop(0, TPB // IBR)
    def _(kb):
        pltpu.sync_copy(gidx.at[c, s, kb], gi_v)
        pltpu.sync_copy(sidx.at[c, s, kb], si_v)
        pltpu.async_copy(tab.at[gi_v.at[0]], buf.at[0], gsem)

        @pl.loop(0, IBR // 2)
        def _(p):
            j0 = 2 * p
            j1 = j0 + 1
            pltpu.make_async_copy(tab.at[gi_v.at[j0]], buf.at[0], gsem).wait()
            pltpu.async_copy(tab.at[gi_v.at[j1]], buf.at[1], gsem)
            pltpu.sync_copy(buf.at[0], acc.at[si_v.at[j0]], add=True)
            pltpu.make_async_copy(tab.at[gi_v.at[j1]], buf.at[1], gsem).wait()

            @pl.when(p < IBR // 2 - 1)
            def _():
                pltpu.async_copy(tab.at[gi_v.at[j0 + 2]], buf.at[0], gsem)

            pltpu.sync_copy(buf.at[1], acc.at[si_v.at[j1]], add=True)

    plsc.subcore_barrier()
    pltpu.sync_copy(acc.at[pl.ds(s * ZR, ZR)], out.at[c, s])


def _mm_body(x_ref, deg_ref, wp_ref, wq_ref, wa_ref, ba_ref, tab_ref, xa_ref):
    xb = x_ref[...]
    d = deg_ref[0]
    r_out = jnp.reciprocal(d[:, 0:1])
    r_in = jnp.reciprocal(d[:, 1:2])
    hi = lax.Precision.HIGHEST
    tab_ref[0] = jnp.dot(xb, wp_ref[...], precision=hi, preferred_element_type=jnp.float32) * r_out
    tab_ref[1] = jnp.dot(xb, wq_ref[...], precision=hi, preferred_element_type=jnp.float32) * r_in
    xa_ref[...] = jnp.dot(xb, wa_ref[...], precision=hi, preferred_element_type=jnp.float32) + ba_ref[...]


def _fin_body(xa_ref, s_ref, wl_ref, bl_ref, o_ref):
    xa = xa_ref[...]
    s_o = s_ref[0]
    s_i = s_ref[1]
    zp = xa[:, :D_HID] + s_o[:, :D_HID] + s_i[:, :D_HID]
    hp = xa[:, D_HID:] + s_o[:, D_HID:] + s_i[:, D_HID:]
    hnew = (1.0 - jax.nn.sigmoid(zp)) * jnp.tanh(hp)
    o_ref[...] = jnp.dot(hnew, wl_ref[...], precision=lax.Precision.HIGHEST, preferred_element_type=jnp.float32) + bl_ref[...]


def kernel(x, edge_index, edge_weight, Wz, bz, Wr, br, Wh, bh, Wl, bl):
    f32 = jnp.float32
    ei = edge_index.astype(jnp.int32)
    row, col = ei[0], ei[1]
    eidx = ei.reshape(2, NTILE, TPB, CHK)
    gidx = jnp.stack([row, col + N]).reshape(2, NTILE, TPB // IBR, IBR, CHK)
    sidx = jnp.stack([col, row]).reshape(2, NTILE, TPB // IBR, IBR, CHK)
    wch = edge_weight.astype(f32).reshape(NTILE, TPB, CHK)
    zn = jnp.zeros((N,), f32)
    zb = jnp.zeros((ZR, D_IN), f32)

    W_P = jnp.concatenate([Wz[0, 1, :D_IN, :], Wh[0, 1, :D_IN, :]], axis=1)
    W_Q = jnp.concatenate([Wz[1, 1, :D_IN, :], Wh[1, 1, :D_IN, :]], axis=1)
    W_A = jnp.concatenate(
        [Wz[0, 0, :D_IN] + Wz[1, 0, :D_IN], Wh[0, 0, :D_IN] + Wh[1, 0, :D_IN]], axis=1
    )
    b_A = jnp.concatenate([bz, bh]).reshape(1, 2 * D_HID)

    mesh = plsc.VectorSubcoreMesh(core_axis_name="c", subcore_axis_name="s")

    deg = pl.kernel(
        _deg_body,
        out_type=jax.ShapeDtypeStruct((2, N), f32),
        mesh=mesh,
        scratch_types=[
            pltpu.VMEM_SHARED((N,), f32),
            pltpu.VMEM((TPB, CHK), jnp.int32),
            pltpu.VMEM((TPB, CHK), f32),
        ],
        name="dcrnn_degrees",
    )(eidx, wch, zn)
    deg_t = deg.T.reshape(NBLK, BR, 2)

    tab3, xa = pl.pallas_call(
        _mm_body,
        grid=(NBLK,),
        in_specs=[
            pl.BlockSpec((BR, D_IN), lambda i: (i, 0)),
            pl.BlockSpec((1, BR, 2), lambda i: (i, 0, 0)),
            pl.BlockSpec((D_IN, D_IN), lambda i: (0, 0)),
            pl.BlockSpec((D_IN, D_IN), lambda i: (0, 0)),
            pl.BlockSpec((D_IN, D_IN), lambda i: (0, 0)),
            pl.BlockSpec((1, 2 * D_HID), lambda i: (0, 0)),
        ],
        out_specs=[
            pl.BlockSpec((2, BR, D_IN), lambda i: (0, i, 0)),
            pl.BlockSpec((BR, D_IN), lambda i: (i, 0)),
        ],
        out_shape=[
            jax.ShapeDtypeStruct((2, N, D_IN), f32),
            jax.ShapeDtypeStruct((N, D_IN), f32),
        ],
        name="dcrnn_tables",
    )(x, deg_t, W_P, W_Q, W_A, b_A)

    tab = tab3.reshape(2 * N, D_IN)

    s_acc = pl.kernel(
        _diff_body,
        out_type=jax.ShapeDtypeStruct((2, NTILE, ZR, D_IN), f32),
        mesh=mesh,
        scratch_types=[
            pltpu.VMEM_SHARED((N, D_IN), f32),
            pltpu.VMEM((IBR, CHK), jnp.int32),
            pltpu.VMEM((IBR, CHK), jnp.int32),
            pltpu.VMEM((NBUF, CHK, D_IN), f32),
            pltpu.SemaphoreType.DMA,
        ],
        name="dcrnn_diffusion",
    )(tab, gidx, sidx, zb)
    s_acc = s_acc.reshape(2, N, D_IN)

    out = pl.pallas_call(
        _fin_body,
        grid=(NBLK,),
        in_specs=[
            pl.BlockSpec((BR, D_IN), lambda i: (i, 0)),
            pl.BlockSpec((2, BR, D_IN), lambda i: (0, i, 0)),
            pl.BlockSpec((D_HID, 1), lambda i: (0, 0)),
            pl.BlockSpec((1, 1), lambda i: (0, 0)),
        ],
        out_specs=pl.BlockSpec((BR, 1), lambda i: (i, 0)),
        out_shape=jax.ShapeDtypeStruct((N, 1), f32),
        name="dcrnn_head",
    )(xa, s_acc, Wl, bl.reshape(1, 1))

    return out[:, 0]


# speedup vs baseline: 2.1425x; 1.1216x over previous
import functools
import jax
import jax.numpy as jnp
from jax import lax
from jax.experimental import pallas as pl
from jax.experimental.pallas import tpu as pltpu
from jax.experimental.pallas import tpu_sc as plsc

N = 10000
E = 320000
D_IN = 128
D_HID = 64

NTILE = 16
CHK = 80
NCH = E // CHK
TPB = NCH // NTILE
NBUF = 4
IBR = 10
ZR = N // NTILE
NBLK = 25
BR = N // NBLK


def _deg_body(eidx, wch, zn, deg_out, deg_s, idx_v, w_v):
    c = lax.axis_index("c")
    s = lax.axis_index("s")

    @pl.when(s == 0)
    def _():
        pltpu.sync_copy(zn, deg_s)

    plsc.subcore_barrier()
    pltpu.sync_copy(eidx.at[c, s], idx_v)
    pltpu.sync_copy(wch.at[s], w_v)

    @pl.loop(0, TPB)
    def _(j):
        pltpu.sync_copy(w_v.at[j], deg_s.at[idx_v.at[j]], add=True)

    plsc.subcore_barrier()

    @pl.when(s == 0)
    def _():
        pltpu.sync_copy(deg_s, deg_out.at[c])


def _diff_body(tab, comb, zb, out, acc, civ, buf, gsem, ssem):
    c = lax.axis_index("c")
    s = lax.axis_index("s")
    pltpu.sync_copy(zb, acc.at[pl.ds(s * ZR, ZR)])
    plsc.subcore_barrier()

    def g_start(j, b):
        pltpu.async_copy(tab.at[civ.at[j]], buf.at[b], gsem.at[b])

    def g_wait(j, b):
        pltpu.make_async_copy(tab.at[civ.at[j]], buf.at[b], gsem.at[b]).wait()

    def s_start(j, b):
        pltpu.async_copy(buf.at[b], acc.at[civ.at[IBR + j]], ssem.at[b], add=True)

    def s_wait(j, b):
        pltpu.make_async_copy(buf.at[b], acc.at[civ.at[IBR + j]], ssem.at[b]).wait()

    @pl.loop(0, TPB // IBR)
    def _(kb):
        pltpu.sync_copy(comb.at[c, s, kb], civ)
        g_start(0, 0)
        g_start(1, 1)
        for j in range(IBR):
            b = j % NBUF
            g_wait(j, b)
            s_start(j, b)
            if j + 2 < IBR:
                if j >= 2:
                    s_wait(j - 2, (j - 2) % NBUF)
                g_start(j + 2, (j + 2) % NBUF)
        for j in range(max(IBR - NBUF, 0), IBR):
            s_wait(j, j % NBUF)

    plsc.subcore_barrier()
    pltpu.sync_copy(acc.at[pl.ds(s * ZR, ZR)], out.at[c, s])


def _mm_body(x_ref, deg_ref, wp_ref, wq_ref, wa_ref, ba_ref, tab_ref, xa_ref):
    xb = x_ref[...]
    d = deg_ref[0]
    r_out = jnp.reciprocal(d[:, 0:1])
    r_in = jnp.reciprocal(d[:, 1:2])
    hi = lax.Precision.HIGHEST
    tab_ref[0] = jnp.dot(xb, wp_ref[...], precision=hi, preferred_element_type=jnp.float32) * r_out
    tab_ref[1] = jnp.dot(xb, wq_ref[...], precision=hi, preferred_element_type=jnp.float32) * r_in
    xa_ref[...] = jnp.dot(xb, wa_ref[...], precision=hi, preferred_element_type=jnp.float32) + ba_ref[...]


def _fin_body(xa_ref, s_ref, wl_ref, bl_ref, o_ref):
    xa = xa_ref[...]
    s_o = s_ref[0]
    s_i = s_ref[1]
    zp = xa[:, :D_HID] + s_o[:, :D_HID] + s_i[:, :D_HID]
    hp = xa[:, D_HID:] + s_o[:, D_HID:] + s_i[:, D_HID:]
    hnew = (1.0 - jax.nn.sigmoid(zp)) * jnp.tanh(hp)
    o_ref[...] = jnp.dot(hnew, wl_ref[...], precision=lax.Precision.HIGHEST, preferred_element_type=jnp.float32) + bl_ref[...]


def kernel(x, edge_index, edge_weight, Wz, bz, Wr, br, Wh, bh, Wl, bl):
    f32 = jnp.float32
    ei = edge_index.astype(jnp.int32)
    row, col = ei[0], ei[1]
    eidx = ei.reshape(2, NTILE, TPB, CHK)
    gidx = jnp.stack([row, col + N]).reshape(2, NTILE, TPB // IBR, IBR, CHK)
    sidx = jnp.stack([col, row]).reshape(2, NTILE, TPB // IBR, IBR, CHK)
    comb = jnp.concatenate([gidx, sidx], axis=3)
    wch = edge_weight.astype(f32).reshape(NTILE, TPB, CHK)
    zn = jnp.zeros((N,), f32)
    zb = jnp.zeros((ZR, D_IN), f32)

    W_P = jnp.concatenate([Wz[0, 1, :D_IN, :], Wh[0, 1, :D_IN, :]], axis=1)
    W_Q = jnp.concatenate([Wz[1, 1, :D_IN, :], Wh[1, 1, :D_IN, :]], axis=1)
    W_A = jnp.concatenate(
        [Wz[0, 0, :D_IN] + Wz[1, 0, :D_IN], Wh[0, 0, :D_IN] + Wh[1, 0, :D_IN]], axis=1
    )
    b_A = jnp.concatenate([bz, bh]).reshape(1, 2 * D_HID)

    mesh = plsc.VectorSubcoreMesh(core_axis_name="c", subcore_axis_name="s")

    deg = pl.kernel(
        _deg_body,
        out_type=jax.ShapeDtypeStruct((2, N), f32),
        mesh=mesh,
        scratch_types=[
            pltpu.VMEM_SHARED((N,), f32),
            pltpu.VMEM((TPB, CHK), jnp.int32),
            pltpu.VMEM((TPB, CHK), f32),
        ],
        name="dcrnn_degrees",
    )(eidx, wch, zn)
    deg_t = deg.T.reshape(NBLK, BR, 2)

    tab3, xa = pl.pallas_call(
        _mm_body,
        grid=(NBLK,),
        in_specs=[
            pl.BlockSpec((BR, D_IN), lambda i: (i, 0)),
            pl.BlockSpec((1, BR, 2), lambda i: (i, 0, 0)),
            pl.BlockSpec((D_IN, D_IN), lambda i: (0, 0)),
            pl.BlockSpec((D_IN, D_IN), lambda i: (0, 0)),
            pl.BlockSpec((D_IN, D_IN), lambda i: (0, 0)),
            pl.BlockSpec((1, 2 * D_HID), lambda i: (0, 0)),
        ],
        out_specs=[
            pl.BlockSpec((2, BR, D_IN), lambda i: (0, i, 0)),
            pl.BlockSpec((BR, D_IN), lambda i: (i, 0)),
        ],
        out_shape=[
            jax.ShapeDtypeStruct((2, N, D_IN), f32),
            jax.ShapeDtypeStruct((N, D_IN), f32),
        ],
        name="dcrnn_tables",
    )(x, deg_t, W_P, W_Q, W_A, b_A)

    tab = tab3.reshape(2 * N, D_IN)

    s_acc = pl.kernel(
        _diff_body,
        out_type=jax.ShapeDtypeStruct((2, NTILE, ZR, D_IN), f32),
        mesh=mesh,
        scratch_types=[
            pltpu.VMEM_SHARED((N, D_IN), f32),
            pltpu.VMEM((2 * IBR, CHK), jnp.int32),
            pltpu.VMEM((NBUF, CHK, D_IN), f32),
            pltpu.SemaphoreType.DMA((NBUF,)),
            pltpu.SemaphoreType.DMA((NBUF,)),
        ],
        name="dcrnn_diffusion",
    )(tab, comb, zb)
    s_acc = s_acc.reshape(2, N, D_IN)

    out = pl.pallas_call(
        _fin_body,
        grid=(NBLK,),
        in_specs=[
            pl.BlockSpec((BR, D_IN), lambda i: (i, 0)),
            pl.BlockSpec((2, BR, D_IN), lambda i: (0, i, 0)),
            pl.BlockSpec((D_HID, 1), lambda i: (0, 0)),
            pl.BlockSpec((1, 1), lambda i: (0, 0)),
        ],
        out_specs=pl.BlockSpec((BR, 1), lambda i: (i, 0)),
        out_shape=jax.ShapeDtypeStruct((N, 1), f32),
        name="dcrnn_head",
    )(xa, s_acc, Wl, bl.reshape(1, 1))

    return out[:, 0]


# speedup vs baseline: 2.2787x; 1.0635x over previous
import functools
import jax
import jax.numpy as jnp
from jax import lax
from jax.experimental import pallas as pl
from jax.experimental.pallas import tpu as pltpu
from jax.experimental.pallas import tpu_sc as plsc

N = 10000
E = 320000
D_IN = 128
D_HID = 64

NTILE = 16
CHK = 80
NCH = E // CHK
TPB = NCH // NTILE
NBUF = 4
IBR = 25
ZR = N // NTILE
NBLK = 25
BR = N // NBLK


def _deg_body(eidx, wch, zn, deg_out, deg_s, idx_v, w_v):
    c = lax.axis_index("c")
    s = lax.axis_index("s")

    @pl.when(s == 0)
    def _():
        pltpu.sync_copy(zn, deg_s)

    plsc.subcore_barrier()
    pltpu.sync_copy(eidx.at[c, s], idx_v)
    pltpu.sync_copy(wch.at[s], w_v)

    @pl.loop(0, TPB)
    def _(j):
        pltpu.sync_copy(w_v.at[j], deg_s.at[idx_v.at[j]], add=True)

    plsc.subcore_barrier()

    @pl.when(s == 0)
    def _():
        pltpu.sync_copy(deg_s, deg_out.at[c])


def _diff_body(tab, comb, zb, out, acc, civ, buf, gsem, ssem):
    c = lax.axis_index("c")
    s = lax.axis_index("s")
    pltpu.sync_copy(zb, acc.at[pl.ds(s * ZR, ZR)])
    plsc.subcore_barrier()

    def g_start(j, b):
        pltpu.async_copy(tab.at[civ.at[j]], buf.at[b], gsem.at[b])

    def g_wait(j, b):
        pltpu.make_async_copy(tab.at[civ.at[j]], buf.at[b], gsem.at[b]).wait()

    def s_start(j, b):
        pltpu.async_copy(buf.at[b], acc.at[civ.at[IBR + j]], ssem.at[b], add=True)

    def s_wait(j, b):
        pltpu.make_async_copy(buf.at[b], acc.at[civ.at[IBR + j]], ssem.at[b]).wait()

    @pl.loop(0, TPB // IBR)
    def _(kb):
        pltpu.sync_copy(comb.at[c, s, kb], civ)
        g_start(0, 0)
        g_start(1, 1)
        for j in range(IBR):
            b = j % NBUF
            g_wait(j, b)
            s_start(j, b)
            if j + 2 < IBR:
                if j >= 2:
                    s_wait(j - 2, (j - 2) % NBUF)
                g_start(j + 2, (j + 2) % NBUF)
        for j in range(max(IBR - NBUF, 0), IBR):
            s_wait(j, j % NBUF)

    plsc.subcore_barrier()
    pltpu.sync_copy(acc.at[pl.ds(s * ZR, ZR)], out.at[c, s])


def _mm_body(x_ref, deg_ref, wp_ref, wq_ref, wa_ref, ba_ref, tab_ref, xa_ref):
    xb = x_ref[...]
    d = deg_ref[0]
    r_out = jnp.reciprocal(d[:, 0:1])
    r_in = jnp.reciprocal(d[:, 1:2])
    hi = lax.Precision.HIGHEST
    tab_ref[0] = jnp.dot(xb, wp_ref[...], precision=hi, preferred_element_type=jnp.float32) * r_out
    tab_ref[1] = jnp.dot(xb, wq_ref[...], precision=hi, preferred_element_type=jnp.float32) * r_in
    xa_ref[...] = jnp.dot(xb, wa_ref[...], precision=hi, preferred_element_type=jnp.float32) + ba_ref[...]


def _fin_body(xa_ref, s_ref, wl_ref, bl_ref, o_ref):
    xa = xa_ref[...]
    s_o = s_ref[0]
    s_i = s_ref[1]
    zp = xa[:, :D_HID] + s_o[:, :D_HID] + s_i[:, :D_HID]
    hp = xa[:, D_HID:] + s_o[:, D_HID:] + s_i[:, D_HID:]
    hnew = (1.0 - jax.nn.sigmoid(zp)) * jnp.tanh(hp)
    o_ref[...] = jnp.dot(hnew, wl_ref[...], precision=lax.Precision.HIGHEST, preferred_element_type=jnp.float32) + bl_ref[...]


def kernel(x, edge_index, edge_weight, Wz, bz, Wr, br, Wh, bh, Wl, bl):
    f32 = jnp.float32
    ei = edge_index.astype(jnp.int32)
    row, col = ei[0], ei[1]
    eidx = ei.reshape(2, NTILE, TPB, CHK)
    gidx = jnp.stack([row, col + N]).reshape(2, NTILE, TPB // IBR, IBR, CHK)
    sidx = jnp.stack([col, row]).reshape(2, NTILE, TPB // IBR, IBR, CHK)
    comb = jnp.concatenate([gidx, sidx], axis=3)
    wch = edge_weight.astype(f32).reshape(NTILE, TPB, CHK)
    zn = jnp.zeros((N,), f32)
    zb = jnp.zeros((ZR, D_IN), f32)

    W_P = jnp.concatenate([Wz[0, 1, :D_IN, :], Wh[0, 1, :D_IN, :]], axis=1)
    W_Q = jnp.concatenate([Wz[1, 1, :D_IN, :], Wh[1, 1, :D_IN, :]], axis=1)
    W_A = jnp.concatenate(
        [Wz[0, 0, :D_IN] + Wz[1, 0, :D_IN], Wh[0, 0, :D_IN] + Wh[1, 0, :D_IN]], axis=1
    )
    b_A = jnp.concatenate([bz, bh]).reshape(1, 2 * D_HID)

    mesh = plsc.VectorSubcoreMesh(core_axis_name="c", subcore_axis_name="s")

    deg = pl.kernel(
        _deg_body,
        out_type=jax.ShapeDtypeStruct((2, N), f32),
        mesh=mesh,
        scratch_types=[
            pltpu.VMEM_SHARED((N,), f32),
            pltpu.VMEM((TPB, CHK), jnp.int32),
            pltpu.VMEM((TPB, CHK), f32),
        ],
        name="dcrnn_degrees",
    )(eidx, wch, zn)
    deg_t = deg.T.reshape(NBLK, BR, 2)

    tab3, xa = pl.pallas_call(
        _mm_body,
        grid=(NBLK,),
        in_specs=[
            pl.BlockSpec((BR, D_IN), lambda i: (i, 0)),
            pl.BlockSpec((1, BR, 2), lambda i: (i, 0, 0)),
            pl.BlockSpec((D_IN, D_IN), lambda i: (0, 0)),
            pl.BlockSpec((D_IN, D_IN), lambda i: (0, 0)),
            pl.BlockSpec((D_IN, D_IN), lambda i: (0, 0)),
            pl.BlockSpec((1, 2 * D_HID), lambda i: (0, 0)),
        ],
        out_specs=[
            pl.BlockSpec((2, BR, D_IN), lambda i: (0, i, 0)),
            pl.BlockSpec((BR, D_IN), lambda i: (i, 0)),
        ],
        out_shape=[
            jax.ShapeDtypeStruct((2, N, D_IN), f32),
            jax.ShapeDtypeStruct((N, D_IN), f32),
        ],
        name="dcrnn_tables",
    )(x, deg_t, W_P, W_Q, W_A, b_A)

    tab = tab3.reshape(2 * N, D_IN)

    s_acc = pl.kernel(
        _diff_body,
        out_type=jax.ShapeDtypeStruct((2, NTILE, ZR, D_IN), f32),
        mesh=mesh,
        scratch_types=[
            pltpu.VMEM_SHARED((N, D_IN), f32),
            pltpu.VMEM((2 * IBR, CHK), jnp.int32),
            pltpu.VMEM((NBUF, CHK, D_IN), f32),
            pltpu.SemaphoreType.DMA((NBUF,)),
            pltpu.SemaphoreType.DMA((NBUF,)),
        ],
        name="dcrnn_diffusion",
    )(tab, comb, zb)
    s_acc = s_acc.reshape(2, N, D_IN)

    out = pl.pallas_call(
        _fin_body,
        grid=(NBLK,),
        in_specs=[
            pl.BlockSpec((BR, D_IN), lambda i: (i, 0)),
            pl.BlockSpec((2, BR, D_IN), lambda i: (0, i, 0)),
            pl.BlockSpec((D_HID, 1), lambda i: (0, 0)),
            pl.BlockSpec((1, 1), lambda i: (0, 0)),
        ],
        out_specs=pl.BlockSpec((BR, 1), lambda i: (i, 0)),
        out_shape=jax.ShapeDtypeStruct((N, 1), f32),
        name="dcrnn_head",
    )(xa, s_acc, Wl, bl.reshape(1, 1))

    return out[:, 0]
